# baseline (speedup 1.0000x reference)
import functools

import jax
import jax.numpy as jnp
from jax import lax
from jax.experimental import pallas as pl
from jax.experimental.pallas import tpu as pltpu
from jax.experimental.pallas import tpu_sc as plsc

_NC = 2
_NS = 16
_L = 16
_CH = 128


def _mm_body(x_ref, w_ref, o_ref):
    o_ref[...] = jnp.dot(x_ref[...], w_ref[...],
                         preferred_element_type=jnp.float32)


def _project(x2d, w, bm):
    m, k = x2d.shape
    n = w.shape[1]
    return pl.pallas_call(
        _mm_body,
        grid=(m // bm,),
        in_specs=[
            pl.BlockSpec((bm, k), lambda i: (i, 0)),
            pl.BlockSpec((k, n), lambda i: (0, 0)),
        ],
        out_specs=pl.BlockSpec((bm, n), lambda i: (i, 0)),
        out_shape=jax.ShapeDtypeStruct((m, n), jnp.float32),
    )(x2d, w)


def _gin_mlp_body(nblk, want_p, p_ref, ma_ref, mb_ref, eps_ref, b1_ref,
                  w2_ref, b2_ref, wn_ref, *outs):
    t = pl.program_id(0)
    i = pl.program_id(1)
    a = (1.0 + eps_ref[0, 0]) * p_ref[...] + ma_ref[...] + mb_ref[...] \
        + b1_ref[...]
    u = jnp.maximum(a, 0.0)
    h = jnp.dot(u, w2_ref[...], preferred_element_type=jnp.float32) \
        + b2_ref[...]
    h = jnp.maximum(h, 0.0)
    if want_p:
        pn_ref, hs_ref = outs
        pn_ref[...] = jnp.dot(h, wn_ref[...],
                              preferred_element_type=jnp.float32)
    else:
        (hs_ref,) = outs
    part = jnp.sum(h, axis=0, keepdims=True)

    @pl.when(jnp.logical_and(t == 0, i == 0))
    def _():
        hs_ref[...] = jnp.zeros_like(hs_ref)

    hs_ref[pl.ds(t, 1), :] += part


def _gin_mlp(p, msg_a, msg_b, eps, b1, w2, b2, w_next, s, n, bm):
    h = p.shape[1]
    want_p = w_next is not None
    nblk = n // bm
    grid = (s, nblk)
    bspec = pl.BlockSpec((bm, h), lambda t, i, nblk=nblk: (t * nblk + i, 0))
    wspec = pl.BlockSpec((h, h), lambda t, i: (0, 0))
    out_shapes = []
    out_specs = []
    if want_p:
        out_shapes.append(jax.ShapeDtypeStruct((s * n, h), jnp.float32))
        out_specs.append(bspec)
    out_shapes.append(jax.ShapeDtypeStruct((s, h), jnp.float32))
    out_specs.append(pl.BlockSpec((s, h), lambda t, i: (0, 0)))
    wn = w_next if want_p else jnp.zeros((h, h), jnp.float32)
    res = pl.pallas_call(
        functools.partial(_gin_mlp_body, nblk, want_p),
        grid=grid,
        in_specs=[
            bspec, bspec, bspec,
            pl.BlockSpec(memory_space=pltpu.SMEM),
            pl.BlockSpec((1, h), lambda t, i: (0, 0)),
            wspec,
            pl.BlockSpec((1, h), lambda t, i: (0, 0)),
            wspec,
        ],
        out_specs=out_specs,
        out_shape=out_shapes,
    )(p, msg_a, msg_b, eps.reshape(1, 1), b1.reshape(1, -1), w2,
      b2.reshape(1, -1), wn)
    if want_p:
        return res[0], res[1]
    return None, res[0]


def _gru_dec_body(s, n, gh, h1s_ref, h2s_ref, wx_ref, wh_ref, bx_ref,
                  bh_ref, wd_ref, bd_ref, z_ref):
    hh = h1s_ref.shape[1]
    h = jnp.zeros((1, gh), jnp.float32)
    for t in range(s):
        e1 = h1s_ref[t:t + 1, :] * (1.0 / n)
        e2 = h2s_ref[t:t + 1, :] * (1.0 / n)
        gx = jnp.dot(e1, wx_ref[0:hh, :],
                     preferred_element_type=jnp.float32) \
            + jnp.dot(e2, wx_ref[hh:2 * hh, :],
                      preferred_element_type=jnp.float32) + bx_ref[...]
        ghv = jnp.dot(h, wh_ref[...],
                      preferred_element_type=jnp.float32) + bh_ref[...]
        r = jax.nn.sigmoid(gx[:, 0:gh] + ghv[:, 0:gh])
        z = jax.nn.sigmoid(gx[:, gh:2 * gh] + ghv[:, gh:2 * gh])
        nn = jnp.tanh(gx[:, 2 * gh:3 * gh] + r * ghv[:, 2 * gh:3 * gh])
        h = (1.0 - z) * nn + z * h
    z_ref[...] = jnp.dot(h, wd_ref[...],
                         preferred_element_type=jnp.float32) + bd_ref[...]


def _gru_decode(h1s, h2s, wx, wh, bx, bh, wd, bd, s, n, gh):
    return pl.pallas_call(
        functools.partial(_gru_dec_body, s, n, gh),
        out_shape=jax.ShapeDtypeStruct((1, n), jnp.float32),
    )(h1s, h2s, wx, wh, bx.reshape(1, -1), bh.reshape(1, -1), wd,
      bd.reshape(1, -1))


def _outer_body(zc_ref, zr_ref, o_ref):
    o_ref[...] = zc_ref[...] * zr_ref[...]


def _outer(zcol, zrow, n, bm):
    return pl.pallas_call(
        _outer_body,
        grid=(n // bm,),
        in_specs=[
            pl.BlockSpec((bm, 1), lambda i: (i, 0)),
            pl.BlockSpec((1, n), lambda i: (0, 0)),
        ],
        out_specs=pl.BlockSpec((bm, n), lambda i: (i, 0)),
        out_shape=jax.ShapeDtypeStruct((n, n), jnp.float32),
    )(zcol, zrow)


def _sc_segsum_body(s, n, e2, h, ptab, rech, zerosh, out,
                    acc, ptab_s, rec_v, rows_v, *sems):
    cid = lax.axis_index("c")
    sid = lax.axis_index("s")
    rpt = -(-(n // _NS) // 8) * 8
    last = n - rpt * (_NS - 1)
    lo = rpt * (_NS - 1)
    ep_core = e2 // _NC
    ep_w = ep_core // _NS
    nch = ep_w // _CH
    tch = e2 // _CH
    semi = sems[:4]
    semg = sems[4:8]
    sems_s = sems[8:12]
    cbase = (cid * ep_core + sid * ep_w) // _CH

    def idx_start(t, c, b):
        pltpu.async_copy(rech.at[t * tch + cbase + c], rec_v.at[b],
                         semi[b])

    def idx_wait(b):
        pltpu.make_async_copy(rech.at[0], rec_v.at[b], semi[b]).wait()

    def gather_start(b):
        pltpu.async_copy(ptab_s.at[rec_v.at[b, 0]], rows_v.at[b],
                         semg[b])

    def gather_wait(b):
        pltpu.make_async_copy(ptab_s.at[rec_v.at[b, 0]], rows_v.at[b],
                              semg[b]).wait()

    gdn = lax.GatherDimensionNumbers(offset_dims=(),
                                     collapsed_slice_dims=(0,),
                                     start_index_map=(0,))

    def scale(b):
        def grp(g, _):
            ea16 = plsc.bitcast(rec_v[b, 2, pl.ds(g * _L, _L)],
                                jnp.float32)
            for j in range(_L):
                r = g * _L + j
                evec = lax.gather(
                    ea16, jnp.full((_L, 1), j, jnp.int32), gdn,
                    slice_sizes=(1,),
                    mode=lax.GatherScatterMode.PROMISE_IN_BOUNDS)
                for f in range(h // _L):
                    rows_v[b, r, pl.ds(f * _L, _L)] = \
                        rows_v[b, r, pl.ds(f * _L, _L)] * evec
            return 0
        lax.fori_loop(0, _CH // _L, grp, 0)

    def scatter_start(k):
        pltpu.async_copy(rows_v.at[k], acc.at[rec_v.at[k, 1]],
                         sems_s[k], add=True)

    def scatter_wait(k):
        pltpu.make_async_copy(rows_v.at[k], acc.at[rec_v.at[k, 1]],
                              sems_s[k]).wait()

    def step(t, chunk, k, nxt1, nxt2, wsc):
        gather_wait(k)
        if nxt1:
            b1 = (k + 1) % 4
            idx_wait(b1)
            gather_start(b1)
        if nxt2:
            b2 = (k + 2) % 4
            if wsc:
                scatter_wait(b2)
            idx_start(t, chunk + 2, b2)
        scale(k)
        scatter_start(k)

    def tbody(t, _):
        @pl.when(sid < _NS - 1)
        def _():
            pltpu.sync_copy(zerosh.at[pl.ds(sid * rpt, rpt)],
                            acc.at[pl.ds(sid * rpt, rpt)])
            pltpu.sync_copy(ptab.at[pl.ds(t * n + sid * rpt, rpt)],
                            ptab_s.at[pl.ds(sid * rpt, rpt)])

        @pl.when(sid == _NS - 1)
        def _():
            pltpu.sync_copy(zerosh.at[pl.ds(lo, last)],
                            acc.at[pl.ds(lo, last)])
            pltpu.sync_copy(ptab.at[pl.ds(t * n + lo, last)],
                            ptab_s.at[pl.ds(lo, last)])

        plsc.subcore_barrier()

        idx_start(t, 0, 0)
        idx_start(t, 1, 1)
        idx_wait(0)
        gather_start(0)

        for k in range(4):
            step(t, k, k, True, True, wsc=(k >= 2))

        def quad(i, _):
            c = 4 * i
            for k in range(4):
                step(t, c + k, k, True, True, wsc=True)
            return 0

        lax.fori_loop(1, nch // 4 - 1, quad, 0)
        for k in range(4):
            chunk = nch - 4 + k
            step(t, chunk, k, chunk + 1 < nch, chunk + 2 < nch, wsc=True)
        for k in range(4):
            scatter_wait(k)

        plsc.subcore_barrier()
        obase = cid * (s * n) + t * n

        @pl.when(sid < _NS - 1)
        def _():
            pltpu.sync_copy(acc.at[pl.ds(sid * rpt, rpt)],
                            out.at[pl.ds(obase + sid * rpt, rpt)])

        @pl.when(sid == _NS - 1)
        def _():
            pltpu.sync_copy(acc.at[pl.ds(lo, last)],
                            out.at[pl.ds(obase + lo, last)])

        plsc.subcore_barrier()
        return 0

    lax.fori_loop(0, s, tbody, 0)


def _segsum(ptab, rec, s, n, e2):
    h = ptab.shape[1]
    zeros = jnp.zeros((n, h), jnp.float32)
    fn = pl.kernel(
        functools.partial(_sc_segsum_body, s, n, e2, h),
        out_type=jax.ShapeDtypeStruct((_NC * s * n, h), jnp.float32),
        mesh=plsc.VectorSubcoreMesh(core_axis_name="c",
                                    subcore_axis_name="s"),
        compiler_params=pltpu.CompilerParams(needs_layout_passes=False,
                                             use_tc_tiling_on_sc=False),
        scratch_types=[
            pltpu.VMEM_SHARED((n, h), jnp.float32),
            pltpu.VMEM_SHARED((n, h), jnp.float32),
            pltpu.VMEM((4, 3, _CH), jnp.int32),
            pltpu.VMEM((4, _CH, h), jnp.float32),
        ] + [pltpu.SemaphoreType.DMA] * 12,
    )
    msg = fn(ptab, rec, zeros)
    return msg[:s * n], msg[s * n:]


def kernel(x, edge_index, edge_attr, W1_0, b1_0, W2_0, b2_0, eps0,
           W1_1, b1_1, W2_1, b2_1, eps1, Wx, Wh, bx, bh, Wd, bd):
    s, n, d = x.shape
    e = edge_index.shape[1]
    gh = Wh.shape[0]
    hdim = W1_0.shape[1]

    src = edge_index[0]
    dst = edge_index[1]
    quant = 4 * _NC * _NS * _CH
    e2 = -(-e // quant) * quant
    padn = e2 - e
    srcp = jnp.concatenate([src, jnp.zeros((padn,), jnp.int32)])
    dstp = jnp.concatenate([dst, jnp.zeros((padn,), jnp.int32)])
    eap = jnp.concatenate(
        [edge_attr.reshape(s, e), jnp.zeros((s, padn), jnp.float32)],
        axis=1)
    tch = e2 // _CH
    rec = jnp.stack([
        jnp.broadcast_to(srcp.reshape(1, tch, _CH), (s, tch, _CH)),
        jnp.broadcast_to(dstp.reshape(1, tch, _CH), (s, tch, _CH)),
        lax.bitcast_convert_type(eap, jnp.int32).reshape(s, tch, _CH),
    ], axis=2).reshape(s * tch, 3, _CH)

    p0 = _project(x.reshape(s * n, d), W1_0, bm=2000)
    m0a, m0b = _segsum(p0, rec, s, n, e2)
    p1, h1s = _gin_mlp(p0, m0a, m0b, eps0, b1_0, W2_0, b2_0, W1_1,
                       s, n, bm=2000)
    m1a, m1b = _segsum(p1, rec, s, n, e2)
    _, h2s = _gin_mlp(p1, m1a, m1b, eps1, b1_1, W2_1, b2_1, None,
                      s, n, bm=2000)
    zdec = _gru_decode(h1s, h2s, Wx, Wh, bx, bh, Wd, bd, s, n, gh)
    return _outer(zdec.reshape(n, 1), zdec, n, bm=400)


# speedup vs baseline: 1.0374x; 1.0374x over previous
import functools

import jax
import jax.numpy as jnp
from jax import lax
from jax.experimental import pallas as pl
from jax.experimental.pallas import tpu as pltpu
from jax.experimental.pallas import tpu_sc as plsc

_NC = 2
_NS = 16
_L = 16
_CH = 128


def _mm_body(x_ref, w_ref, o_ref):
    o_ref[...] = jnp.dot(x_ref[...], w_ref[...],
                         preferred_element_type=jnp.float32)


def _project(x2d, w, bm):
    m, k = x2d.shape
    n = w.shape[1]
    return pl.pallas_call(
        _mm_body,
        grid=(m // bm,),
        in_specs=[
            pl.BlockSpec((bm, k), lambda i: (i, 0)),
            pl.BlockSpec((k, n), lambda i: (0, 0)),
        ],
        out_specs=pl.BlockSpec((bm, n), lambda i: (i, 0)),
        out_shape=jax.ShapeDtypeStruct((m, n), jnp.float32),
    )(x2d, w)


def _gin_mlp_body(nblk, want_p, p_ref, ma_ref, mb_ref, eps_ref, b1_ref,
                  w2_ref, b2_ref, wn_ref, *outs):
    t = pl.program_id(0)
    i = pl.program_id(1)
    a = (1.0 + eps_ref[0, 0]) * p_ref[...] + ma_ref[...] + mb_ref[...] \
        + b1_ref[...]
    u = jnp.maximum(a, 0.0)
    h = jnp.dot(u, w2_ref[...], preferred_element_type=jnp.float32) \
        + b2_ref[...]
    h = jnp.maximum(h, 0.0)
    if want_p:
        pn_ref, hs_ref = outs
        pn_ref[...] = jnp.dot(h, wn_ref[...],
                              preferred_element_type=jnp.float32)
    else:
        (hs_ref,) = outs
    part = jnp.sum(h, axis=0, keepdims=True)

    @pl.when(jnp.logical_and(t == 0, i == 0))
    def _():
        hs_ref[...] = jnp.zeros_like(hs_ref)

    hs_ref[pl.ds(t, 1), :] += part


def _gin_mlp(p, msg_a, msg_b, eps, b1, w2, b2, w_next, s, n, bm):
    h = p.shape[1]
    want_p = w_next is not None
    nblk = n // bm
    grid = (s, nblk)
    bspec = pl.BlockSpec((bm, h), lambda t, i, nblk=nblk: (t * nblk + i, 0))
    wspec = pl.BlockSpec((h, h), lambda t, i: (0, 0))
    out_shapes = []
    out_specs = []
    if want_p:
        out_shapes.append(jax.ShapeDtypeStruct((s * n, h), jnp.float32))
        out_specs.append(bspec)
    out_shapes.append(jax.ShapeDtypeStruct((s, h), jnp.float32))
    out_specs.append(pl.BlockSpec((s, h), lambda t, i: (0, 0)))
    wn = w_next if want_p else jnp.zeros((h, h), jnp.float32)
    res = pl.pallas_call(
        functools.partial(_gin_mlp_body, nblk, want_p),
        grid=grid,
        in_specs=[
            bspec, bspec, bspec,
            pl.BlockSpec(memory_space=pltpu.SMEM),
            pl.BlockSpec((1, h), lambda t, i: (0, 0)),
            wspec,
            pl.BlockSpec((1, h), lambda t, i: (0, 0)),
            wspec,
        ],
        out_specs=out_specs,
        out_shape=out_shapes,
    )(p, msg_a, msg_b, eps.reshape(1, 1), b1.reshape(1, -1), w2,
      b2.reshape(1, -1), wn)
    if want_p:
        return res[0], res[1]
    return None, res[0]


def _gru_dec_body(s, n, gh, h1s_ref, h2s_ref, wx_ref, wh_ref, bx_ref,
                  bh_ref, wd_ref, bd_ref, z_ref):
    hh = h1s_ref.shape[1]
    h = jnp.zeros((1, gh), jnp.float32)
    for t in range(s):
        e1 = h1s_ref[t:t + 1, :] * (1.0 / n)
        e2 = h2s_ref[t:t + 1, :] * (1.0 / n)
        gx = jnp.dot(e1, wx_ref[0:hh, :],
                     preferred_element_type=jnp.float32) \
            + jnp.dot(e2, wx_ref[hh:2 * hh, :],
                      preferred_element_type=jnp.float32) + bx_ref[...]
        ghv = jnp.dot(h, wh_ref[...],
                      preferred_element_type=jnp.float32) + bh_ref[...]
        r = jax.nn.sigmoid(gx[:, 0:gh] + ghv[:, 0:gh])
        z = jax.nn.sigmoid(gx[:, gh:2 * gh] + ghv[:, gh:2 * gh])
        nn = jnp.tanh(gx[:, 2 * gh:3 * gh] + r * ghv[:, 2 * gh:3 * gh])
        h = (1.0 - z) * nn + z * h
    z_ref[...] = jnp.dot(h, wd_ref[...],
                         preferred_element_type=jnp.float32) + bd_ref[...]


def _gru_decode(h1s, h2s, wx, wh, bx, bh, wd, bd, s, n, gh):
    return pl.pallas_call(
        functools.partial(_gru_dec_body, s, n, gh),
        out_shape=jax.ShapeDtypeStruct((1, n), jnp.float32),
    )(h1s, h2s, wx, wh, bx.reshape(1, -1), bh.reshape(1, -1), wd,
      bd.reshape(1, -1))


def _outer_body(zc_ref, zr_ref, o_ref):
    o_ref[...] = zc_ref[...] * zr_ref[...]


def _outer(zcol, zrow, n, bm):
    return pl.pallas_call(
        _outer_body,
        grid=(n // bm,),
        in_specs=[
            pl.BlockSpec((bm, 1), lambda i: (i, 0)),
            pl.BlockSpec((1, n), lambda i: (0, 0)),
        ],
        out_specs=pl.BlockSpec((bm, n), lambda i: (i, 0)),
        out_shape=jax.ShapeDtypeStruct((n, n), jnp.float32),
    )(zcol, zrow)


def _sc_segsum_body(s, n, e2, h, ptab, srcoff, dsth, eah, zerosh, out,
                    acc, ptab_s, src_v, dst_v, ea_v, rows_v, *sems):
    cid = lax.axis_index("c")
    sid = lax.axis_index("s")
    rpt = -(-(n // _NS) // 8) * 8
    last = n - rpt * (_NS - 1)
    lo = rpt * (_NS - 1)
    ep_core = e2 // _NC
    ep_w = ep_core // _NS
    nch = ep_w // _CH
    semi = sems[:4]
    semg = sems[4:8]
    sems_s = sems[8:12]
    ebase = cid * ep_core + sid * ep_w

    def idx_start(t, c, b):
        o = ebase + c * _CH
        pltpu.async_copy(srcoff.at[pl.ds(o, _CH)], src_v.at[b], semi[b])
        pltpu.async_copy(eah.at[pl.ds(t * e2 + o, _CH)], ea_v.at[b],
                         semi[b])
        pltpu.async_copy(dsth.at[pl.ds(o, _CH)], dst_v.at[b], semi[b])

    def idx_wait(b):
        pltpu.make_async_copy(srcoff.at[pl.ds(0, _CH)], src_v.at[b],
                              semi[b]).wait()
        pltpu.make_async_copy(eah.at[pl.ds(0, _CH)], ea_v.at[b],
                              semi[b]).wait()
        pltpu.make_async_copy(dsth.at[pl.ds(0, _CH)], dst_v.at[b],
                              semi[b]).wait()

    def gather_start(b):
        pltpu.async_copy(ptab_s.at[src_v.at[b]], rows_v.at[b], semg[b])

    def gather_wait(b):
        pltpu.make_async_copy(ptab_s.at[src_v.at[b]], rows_v.at[b],
                              semg[b]).wait()

    gdn = lax.GatherDimensionNumbers(offset_dims=(),
                                     collapsed_slice_dims=(0,),
                                     start_index_map=(0,))

    def scale(b):
        def grp(g, _):
            ea16 = ea_v[b, pl.ds(g * _L, _L)]
            for j in range(_L):
                r = g * _L + j
                evec = lax.gather(
                    ea16, jnp.full((_L, 1), j, jnp.int32), gdn,
                    slice_sizes=(1,),
                    mode=lax.GatherScatterMode.PROMISE_IN_BOUNDS)
                for f in range(h // _L):
                    rows_v[b, r, pl.ds(f * _L, _L)] = \
                        rows_v[b, r, pl.ds(f * _L, _L)] * evec
            return 0
        lax.fori_loop(0, _CH // _L, grp, 0)

    def scatter_start(k):
        pltpu.async_copy(rows_v.at[k], acc.at[dst_v.at[k]], sems_s[k],
                         add=True)

    def scatter_wait(k):
        pltpu.make_async_copy(rows_v.at[k], acc.at[dst_v.at[k]],
                              sems_s[k]).wait()

    def step(t, chunk, k, nxt1, nxt2, wsc):
        gather_wait(k)
        if nxt1:
            b1 = (k + 1) % 4
            idx_wait(b1)
            gather_start(b1)
        if nxt2:
            b2 = (k + 2) % 4
            if wsc:
                scatter_wait(b2)
            idx_start(t, chunk + 2, b2)
        scale(k)
        scatter_start(k)

    def tbody(t, _):
        @pl.when(sid < _NS - 1)
        def _():
            pltpu.sync_copy(zerosh.at[pl.ds(sid * rpt, rpt)],
                            acc.at[pl.ds(sid * rpt, rpt)])
            pltpu.sync_copy(ptab.at[pl.ds(t * n + sid * rpt, rpt)],
                            ptab_s.at[pl.ds(sid * rpt, rpt)])

        @pl.when(sid == _NS - 1)
        def _():
            pltpu.sync_copy(zerosh.at[pl.ds(lo, last)],
                            acc.at[pl.ds(lo, last)])
            pltpu.sync_copy(ptab.at[pl.ds(t * n + lo, last)],
                            ptab_s.at[pl.ds(lo, last)])

        plsc.subcore_barrier()

        idx_start(t, 0, 0)
        idx_start(t, 1, 1)
        idx_wait(0)
        gather_start(0)

        for k in range(4):
            step(t, k, k, True, True, wsc=(k >= 2))

        def quad(i, _):
            c = 4 * i
            for k in range(4):
                step(t, c + k, k, True, True, wsc=True)
            return 0

        lax.fori_loop(1, nch // 4 - 1, quad, 0)
        for k in range(4):
            chunk = nch - 4 + k
            step(t, chunk, k, chunk + 1 < nch, chunk + 2 < nch, wsc=True)
        for k in range(4):
            scatter_wait(k)

        plsc.subcore_barrier()
        obase = cid * (s * n) + t * n

        @pl.when(sid < _NS - 1)
        def _():
            pltpu.sync_copy(acc.at[pl.ds(sid * rpt, rpt)],
                            out.at[pl.ds(obase + sid * rpt, rpt)])

        @pl.when(sid == _NS - 1)
        def _():
            pltpu.sync_copy(acc.at[pl.ds(lo, last)],
                            out.at[pl.ds(obase + lo, last)])

        plsc.subcore_barrier()
        return 0

    lax.fori_loop(0, s, tbody, 0)


def _segsum(ptab, src_off, dst, ea_flat, s, n, e2):
    h = ptab.shape[1]
    zeros = jnp.zeros((n, h), jnp.float32)
    fn = pl.kernel(
        functools.partial(_sc_segsum_body, s, n, e2, h),
        out_type=jax.ShapeDtypeStruct((_NC * s * n, h), jnp.float32),
        mesh=plsc.VectorSubcoreMesh(core_axis_name="c",
                                    subcore_axis_name="s"),
        compiler_params=pltpu.CompilerParams(needs_layout_passes=False,
                                             use_tc_tiling_on_sc=False),
        scratch_types=[
            pltpu.VMEM_SHARED((n, h), jnp.float32),
            pltpu.VMEM_SHARED((n, h), jnp.float32),
            pltpu.VMEM((4, _CH), jnp.int32),
            pltpu.VMEM((4, _CH), jnp.int32),
            pltpu.VMEM((4, _CH), jnp.float32),
            pltpu.VMEM((4, _CH, h), jnp.float32),
        ] + [pltpu.SemaphoreType.DMA] * 12,
    )
    msg = fn(ptab, src_off, dst, ea_flat, zeros)
    return msg[:s * n], msg[s * n:]


def kernel(x, edge_index, edge_attr, W1_0, b1_0, W2_0, b2_0, eps0,
           W1_1, b1_1, W2_1, b2_1, eps1, Wx, Wh, bx, bh, Wd, bd):
    s, n, d = x.shape
    e = edge_index.shape[1]
    gh = Wh.shape[0]
    hdim = W1_0.shape[1]

    src = edge_index[0]
    dst = edge_index[1]
    quant = 4 * _NC * _NS * _CH
    e2 = -(-e // quant) * quant
    padn = e2 - e
    srcp = jnp.concatenate([src, jnp.zeros((padn,), jnp.int32)])
    dstp = jnp.concatenate([dst, jnp.zeros((padn,), jnp.int32)])
    ea_flat = jnp.concatenate(
        [edge_attr.reshape(s, e), jnp.zeros((s, padn), jnp.float32)],
        axis=1).reshape(-1)

    p0 = _project(x.reshape(s * n, d), W1_0, bm=2000)
    m0a, m0b = _segsum(p0, srcp, dstp, ea_flat, s, n, e2)
    p1, h1s = _gin_mlp(p0, m0a, m0b, eps0, b1_0, W2_0, b2_0, W1_1,
                       s, n, bm=2000)
    m1a, m1b = _segsum(p1, srcp, dstp, ea_flat, s, n, e2)
    _, h2s = _gin_mlp(p1, m1a, m1b, eps1, b1_1, W2_1, b2_1, None,
                      s, n, bm=2000)
    zdec = _gru_decode(h1s, h2s, Wx, Wh, bx, bh, Wd, bd, s, n, gh)
    return _outer(zdec.reshape(n, 1), zdec, n, bm=400)
